# baseline (speedup 1.0000x reference)
import functools

import jax
import jax.numpy as jnp
from jax.experimental import pallas as pl
from jax.experimental.pallas import tpu as pltpu


def _adapter_block(x_ref, w1_ref, r1_ref, b1_ref, w2_ref, b2_ref,
                   ln2_g_ref, ln2_b_ref, o_ref):
    x = x_ref[...]
    xf = x.astype(jnp.float32)
    m = jnp.mean(xf, axis=-1, keepdims=True)
    ex2 = jnp.mean(xf * xf, axis=-1, keepdims=True)
    s = jax.lax.rsqrt(ex2 - m * m + 1e-5)
    y = jnp.dot(x, w1_ref[...], preferred_element_type=jnp.float32)
    h1 = s * y - (s * m) * r1_ref[...] + b1_ref[...]
    g = 0.5 * h1 * (1.0 + jax.lax.erf(h1 * 0.7071067811865476))
    h2 = jnp.dot(g.astype(jnp.bfloat16), w2_ref[...],
                 preferred_element_type=jnp.float32) + b2_ref[...]
    m2 = jnp.mean(h2, axis=-1, keepdims=True)
    e2 = jnp.mean(h2 * h2, axis=-1, keepdims=True)
    s2 = jax.lax.rsqrt(e2 - m2 * m2 + 1e-5)
    o_ref[...] = (h2 - m2) * (s2 * ln2_g_ref[...]) + ln2_b_ref[...]


@functools.partial(jax.jit, static_argnames=("tm",))
def _run(xb, w1b, r1, b1, w2b, b2, ln2_g, ln2_b, tm):
    n, d = xb.shape
    h = w1b.shape[1]
    o = w2b.shape[1]
    grid = (n // tm,)
    const = lambda i: (0, 0)
    out = pl.pallas_call(
        _adapter_block,
        grid=grid,
        in_specs=[
            pl.BlockSpec((tm, d), lambda i: (i, 0)),
            pl.BlockSpec((d, h), const),
            pl.BlockSpec((1, h), const),
            pl.BlockSpec((1, h), const),
            pl.BlockSpec((h, o), const),
            pl.BlockSpec((1, o), const),
            pl.BlockSpec((1, o), const),
            pl.BlockSpec((1, o), const),
        ],
        out_specs=pl.BlockSpec((tm, o), lambda i: (i, 0)),
        out_shape=jax.ShapeDtypeStruct((n, o), jnp.float32),
        compiler_params=pltpu.CompilerParams(
            dimension_semantics=("arbitrary",),
        ),
    )(xb, w1b, r1, b1, w2b, b2, ln2_g, ln2_b)
    return out


def kernel(x, ln_g, ln_b, W1, b1, W2, b2, ln2_g, ln2_b):
    B, T, D = x.shape
    H = W1.shape[1]
    O = W2.shape[1]
    xb = x.reshape(B * T, D).astype(jnp.bfloat16)
    w1b = (ln_g[:, None] * W1).astype(jnp.bfloat16)
    r1 = jnp.sum(w1b.astype(jnp.float32), axis=0).reshape(1, H)
    b1f = (ln_b @ W1 + b1).reshape(1, H)
    out = _run(xb, w1b, r1, b1f,
               W2.astype(jnp.bfloat16), b2.reshape(1, O),
               ln2_g.reshape(1, O), ln2_b.reshape(1, O),
               tm=512)
    return out.reshape(B, T, O)


# speedup vs baseline: 1.3076x; 1.3076x over previous
import functools

import jax
import jax.numpy as jnp
from jax.experimental import pallas as pl
from jax.experimental.pallas import tpu as pltpu


def _adapter_block(x_ref, g_ref, gb_ref, w1_ref, b1_ref, w2_ref, b2_ref,
                   ln2_g_ref, ln2_b_ref, o_ref, w2b_ref, aux_ref):
    @pl.when(pl.program_id(0) == 0)
    def _prep():
        w2b_ref[...] = w2_ref[...].astype(jnp.bfloat16)
        aux_ref[...] = jnp.dot(gb_ref[...], w1_ref[...],
                               preferred_element_type=jnp.float32)

    x = x_ref[...]
    m = jnp.mean(x, axis=-1, keepdims=True)
    ex2 = jnp.mean(x * x, axis=-1, keepdims=True)
    s = jax.lax.rsqrt(ex2 - m * m + 1e-5)
    sm = s * m
    xg = (x * g_ref[...]).astype(jnp.bfloat16)
    y = jnp.dot(xg, w1_ref[...], preferred_element_type=jnp.float32)
    r1 = aux_ref[0:1, :]
    cvec = aux_ref[1:2, :] + b1_ref[...]
    hdim = y.shape[1]
    hc = hdim // 2
    h2 = None
    for j in range(2):
        h1 = s * y[:, j * hc:(j + 1) * hc] \
            - sm * r1[:, j * hc:(j + 1) * hc] \
            + cvec[:, j * hc:(j + 1) * hc]
        gl = 0.5 * h1 * (1.0 + jax.lax.erf(h1 * 0.7071067811865476))
        p = jnp.dot(gl.astype(jnp.bfloat16), w2b_ref[j * hc:(j + 1) * hc, :],
                    preferred_element_type=jnp.float32)
        h2 = p if h2 is None else h2 + p
    h2 = h2 + b2_ref[...]
    m2 = jnp.mean(h2, axis=-1, keepdims=True)
    e2 = jnp.mean(h2 * h2, axis=-1, keepdims=True)
    s2 = jax.lax.rsqrt(e2 - m2 * m2 + 1e-5)
    o_ref[...] = (h2 - m2) * (s2 * ln2_g_ref[...]) + ln2_b_ref[...]


@functools.partial(jax.jit, static_argnames=("tm",))
def _run(x2d, g_row, gb, w1, b1, w2, b2, ln2_g, ln2_b, tm):
    n, d = x2d.shape
    h = w1.shape[1]
    o = w2.shape[1]
    grid = (n // tm,)
    const = lambda i: (0, 0)
    out = pl.pallas_call(
        _adapter_block,
        grid=grid,
        in_specs=[
            pl.BlockSpec((tm, d), lambda i: (i, 0)),
            pl.BlockSpec((1, d), const),
            pl.BlockSpec((8, d), const),
            pl.BlockSpec((d, h), const),
            pl.BlockSpec((1, h), const),
            pl.BlockSpec((h, o), const),
            pl.BlockSpec((1, o), const),
            pl.BlockSpec((1, o), const),
            pl.BlockSpec((1, o), const),
        ],
        out_specs=pl.BlockSpec((tm, o), lambda i: (i, 0)),
        out_shape=jax.ShapeDtypeStruct((n, o), jnp.float32),
        scratch_shapes=[
            pltpu.VMEM((h, o), jnp.bfloat16),
            pltpu.VMEM((8, h), jnp.float32),
        ],
        compiler_params=pltpu.CompilerParams(
            dimension_semantics=("arbitrary",),
            vmem_limit_bytes=100 * 1024 * 1024,
        ),
    )(x2d, g_row, gb, w1, b1, w2, b2, ln2_g, ln2_b)
    return out


def kernel(x, ln_g, ln_b, W1, b1, W2, b2, ln2_g, ln2_b):
    B, T, D = x.shape
    H = W1.shape[1]
    O = W2.shape[1]
    x2d = x.reshape(B * T, D)
    gb = jnp.zeros((8, D), jnp.bfloat16)
    gb = gb.at[0, :].set(ln_g.astype(jnp.bfloat16))
    gb = gb.at[1, :].set(ln_b.astype(jnp.bfloat16))
    out = _run(x2d, ln_g.reshape(1, D), gb,
               W1.astype(jnp.bfloat16), b1.reshape(1, H),
               W2, b2.reshape(1, O),
               ln2_g.reshape(1, O), ln2_b.reshape(1, O),
               tm=512)
    return out.reshape(B, T, O)
